# baseline (speedup 1.0000x reference)
import functools

import jax
import jax.numpy as jnp
from jax import lax
from jax.experimental import pallas as pl
from jax.experimental.pallas import tpu as pltpu
from jax.experimental.pallas import tpu_sc as plsc

D_MODEL = 2048
NUM_EXPERTS = 8
EXPERT_DIM = 1024
SHARED_DIM = 2048
BR = 256


def _sigmoid(z):
    return 1.0 / (1.0 + jnp.exp(-z))


def _router_body(x_ref, rwt_ref, w_ref):
    xf = x_ref[...]
    logits = jnp.dot(xf, rwt_ref[...], preferred_element_type=jnp.float32)
    iota = jax.lax.broadcasted_iota(jnp.int32, logits.shape, 1)
    big = jnp.int32(2**30)
    m1 = jnp.max(logits, axis=1, keepdims=True)
    a1 = jnp.min(jnp.where(logits == m1, iota, big), axis=1, keepdims=True)
    masked = jnp.where(iota == a1, -jnp.inf, logits)
    m2 = jnp.max(masked, axis=1, keepdims=True)
    a2 = jnp.min(jnp.where(masked == m2, iota, big), axis=1, keepdims=True)
    w_ref[...] = jnp.where(iota == a1, _sigmoid(m1), 0.0) + jnp.where(
        iota == a2, _sigmoid(m2), 0.0)


def _expert_body(be_ref, disp_ref, guw_ref, dw_ref, out_ref):
    del be_ref
    xb = disp_ref[...]
    gu = jnp.dot(xb, guw_ref[0], preferred_element_type=jnp.float32)
    act = _sigmoid(gu[:, :EXPERT_DIM]) * gu[:, :EXPERT_DIM] * gu[:, EXPERT_DIM:]
    out_ref[...] = jnp.dot(act.astype(jnp.bfloat16), dw_ref[0],
                           preferred_element_type=jnp.float32).astype(jnp.bfloat16)


def _combine_body(x_ref, sguw_ref, sdwt_ref, b0_ref, b1_ref, w1_ref, w2_ref,
                  out_ref):
    xb = x_ref[...].astype(jnp.bfloat16)
    gu = jnp.dot(xb, sguw_ref[...], preferred_element_type=jnp.float32)
    act = _sigmoid(gu[:, :SHARED_DIM]) * gu[:, :SHARED_DIM] * gu[:, SHARED_DIM:]
    outs = jnp.dot(act.astype(jnp.bfloat16), sdwt_ref[...],
                   preferred_element_type=jnp.float32)
    out_ref[...] = (outs + w1_ref[...] * b0_ref[...].astype(jnp.float32)
                    + w2_ref[...] * b1_ref[...].astype(jnp.float32))


def _make_row_gather(n_rows_table, n_rows_out, n_cols):
    info = plsc.get_sparse_core_info()
    nw = info.num_cores * info.num_subcores
    b_per_w = n_rows_out // nw
    ch = 32
    while b_per_w % ch:
        ch //= 2
    nch = b_per_w // ch
    mesh = plsc.VectorSubcoreMesh(core_axis_name="c", subcore_axis_name="s")

    @functools.partial(
        pl.kernel,
        out_type=jax.ShapeDtypeStruct((n_rows_out, n_cols), jnp.int32),
        mesh=mesh,
        scratch_types=[
            pltpu.VMEM((b_per_w,), jnp.int32),
            pltpu.VMEM((ch, n_cols), jnp.int32),
            pltpu.VMEM((ch, n_cols), jnp.int32),
            pltpu.SemaphoreType.DMA,
            pltpu.SemaphoreType.DMA,
        ],
    )
    def gather(table_hbm, idx_hbm, out_hbm, idx_v, rows0, rows1, sem0, sem1):
        wid = lax.axis_index("s") * info.num_cores + lax.axis_index("c")
        base = wid * b_per_w
        pltpu.sync_copy(idx_hbm.at[pl.ds(base, b_per_w)], idx_v)
        bufs = (rows0, rows1)
        sems = (sem0, sem1)
        handles = [None] * nch
        handles[0] = pltpu.async_copy(
            table_hbm.at[idx_v.at[pl.ds(0, ch)]], bufs[0], sems[0])
        for c in range(nch):
            if c + 1 < nch:
                handles[c + 1] = pltpu.async_copy(
                    table_hbm.at[idx_v.at[pl.ds((c + 1) * ch, ch)]],
                    bufs[(c + 1) % 2], sems[(c + 1) % 2])
            handles[c].wait()
            pltpu.sync_copy(bufs[c % 2], out_hbm.at[pl.ds(base + c * ch, ch)])

    return gather


def kernel(x, router_w, gate_up_w, down_w, shared_gate_w, shared_up_w,
           shared_down_w):
    B, T, C = x.shape
    N = B * T
    P = N * 2
    P_max = P + NUM_EXPERTS * BR
    G = P_max // BR
    x_flat = x.reshape(N, C)
    rwt = router_w.T
    guw16 = gate_up_w.astype(jnp.bfloat16)
    dw16 = down_w.astype(jnp.bfloat16)
    sguw = jnp.concatenate([shared_gate_w.T, shared_up_w.T], axis=1).astype(jnp.bfloat16)
    sdwt = shared_down_w.T.astype(jnp.bfloat16)

    BTR = 512
    w_dense = pl.pallas_call(
        _router_body,
        grid=(N // BTR,),
        in_specs=[
            pl.BlockSpec((BTR, C), lambda i: (i, 0)),
            pl.BlockSpec((C, NUM_EXPERTS), lambda i: (0, 0)),
        ],
        out_specs=pl.BlockSpec((BTR, NUM_EXPERTS), lambda i: (i, 0)),
        out_shape=jax.ShapeDtypeStruct((N, NUM_EXPERTS), jnp.float32),
    )(x_flat, rwt)

    eye = jnp.arange(NUM_EXPERTS, dtype=jnp.int32)
    w1 = jnp.max(w_dense, axis=1)
    e1 = jnp.argmax(w_dense, axis=1).astype(jnp.int32)
    wd2 = jnp.where(eye[None, :] == e1[:, None], -1.0, w_dense)
    w2 = jnp.max(wd2, axis=1)
    e2 = jnp.argmax(wd2, axis=1).astype(jnp.int32)
    sel = jnp.stack([e1, e2], axis=1).reshape(-1)
    order = jnp.argsort(sel, stable=True).astype(jnp.int32)
    sorted_e = sel[order]
    counts = jnp.sum((sel[:, None] == eye[None, :]).astype(jnp.int32), axis=0)
    offsets = jnp.concatenate([jnp.zeros(1, jnp.int32), jnp.cumsum(counts)[:-1]])
    pcounts = ((counts + BR - 1) // BR) * BR
    pcum = jnp.cumsum(pcounts)
    poffsets = jnp.concatenate([jnp.zeros(1, jnp.int32), pcum[:-1]])
    shift = (poffsets - offsets).astype(jnp.int32)
    pos_sorted = jnp.arange(P, dtype=jnp.int32) + shift[sorted_e]
    tok_padded = jnp.zeros(P_max, jnp.int32).at[pos_sorted].set(
        order // 2, mode="drop")
    block_expert = jnp.minimum(
        jnp.sum((jnp.arange(G, dtype=jnp.int32)[:, None] * BR >= pcum[None, :])
                .astype(jnp.int32), axis=1),
        NUM_EXPERTS - 1).astype(jnp.int32)
    inv = jnp.argsort(order).astype(jnp.int32)
    pos_unsorted = inv + shift[sel]
    p_cat = jnp.concatenate([pos_unsorted[0::2], pos_unsorted[1::2]])

    x16_i32 = lax.bitcast_convert_type(
        x_flat.astype(jnp.bfloat16).reshape(N, C // 2, 2), jnp.int32)
    dispatch = lax.bitcast_convert_type(
        _make_row_gather(N, P_max, C // 2)(x16_i32, tok_padded),
        jnp.bfloat16).reshape(P_max, C)

    grid_spec = pltpu.PrefetchScalarGridSpec(
        num_scalar_prefetch=1,
        grid=(G,),
        in_specs=[
            pl.BlockSpec((BR, C), lambda g, be: (g, 0)),
            pl.BlockSpec((1, C, 2 * EXPERT_DIM), lambda g, be: (be[g], 0, 0)),
            pl.BlockSpec((1, EXPERT_DIM, C), lambda g, be: (be[g], 0, 0)),
        ],
        out_specs=pl.BlockSpec((BR, C), lambda g, be: (g, 0)),
    )
    out_sorted = pl.pallas_call(
        _expert_body,
        grid_spec=grid_spec,
        out_shape=jax.ShapeDtypeStruct((P_max, C), jnp.bfloat16),
        compiler_params=pltpu.CompilerParams(
            dimension_semantics=("arbitrary",)),
    )(block_expert, dispatch, guw16, dw16)

    out_sorted_i32 = lax.bitcast_convert_type(
        out_sorted.reshape(P_max, C // 2, 2), jnp.int32)
    bufs = lax.bitcast_convert_type(
        _make_row_gather(P_max, P, C // 2)(out_sorted_i32, p_cat),
        jnp.bfloat16).reshape(P, C)

    BT2 = 256
    nb2 = N // BT2
    out = pl.pallas_call(
        _combine_body,
        grid=(nb2,),
        in_specs=[
            pl.BlockSpec((BT2, C), lambda i: (i, 0)),
            pl.BlockSpec((C, 2 * SHARED_DIM), lambda i: (0, 0)),
            pl.BlockSpec((SHARED_DIM, C), lambda i: (0, 0)),
            pl.BlockSpec((BT2, C), lambda i: (i, 0)),
            pl.BlockSpec((BT2, C), lambda i, _n=nb2: (i + _n, 0)),
            pl.BlockSpec((BT2, 1), lambda i: (i, 0)),
            pl.BlockSpec((BT2, 1), lambda i: (i, 0)),
        ],
        out_specs=pl.BlockSpec((BT2, C), lambda i: (i, 0)),
        out_shape=jax.ShapeDtypeStruct((N, C), jnp.float32),
    )(x_flat, sguw, sdwt, bufs, bufs, w1.reshape(N, 1), w2.reshape(N, 1))
    return out.reshape(B, T, C)


# speedup vs baseline: 2.6483x; 2.6483x over previous
import functools

import jax
import jax.numpy as jnp
from jax import lax
from jax.experimental import pallas as pl
from jax.experimental.pallas import tpu as pltpu
from jax.experimental.pallas import tpu_sc as plsc

D_MODEL = 2048
NUM_EXPERTS = 8
EXPERT_DIM = 1024
SHARED_DIM = 2048
BR = 256


def _sigmoid(z):
    return 1.0 / (1.0 + jnp.exp(-z))


def _router_body(x_ref, rwt_ref, w_ref):
    xf = x_ref[...]
    logits = jnp.dot(xf, rwt_ref[...], preferred_element_type=jnp.float32)
    iota = jax.lax.broadcasted_iota(jnp.int32, logits.shape, 1)
    big = jnp.int32(2**30)
    m1 = jnp.max(logits, axis=1, keepdims=True)
    a1 = jnp.min(jnp.where(logits == m1, iota, big), axis=1, keepdims=True)
    masked = jnp.where(iota == a1, -jnp.inf, logits)
    m2 = jnp.max(masked, axis=1, keepdims=True)
    a2 = jnp.min(jnp.where(masked == m2, iota, big), axis=1, keepdims=True)
    w_ref[...] = jnp.where(iota == a1, _sigmoid(m1), 0.0) + jnp.where(
        iota == a2, _sigmoid(m2), 0.0)


def _expert_body(be_ref, disp_ref, guw_ref, dw_ref, out_ref):
    del be_ref
    xb = disp_ref[...].astype(jnp.bfloat16)
    gu = jnp.dot(xb, guw_ref[0], preferred_element_type=jnp.float32)
    act = _sigmoid(gu[:, :EXPERT_DIM]) * gu[:, :EXPERT_DIM] * gu[:, EXPERT_DIM:]
    out_ref[...] = jnp.dot(act.astype(jnp.bfloat16), dw_ref[0],
                           preferred_element_type=jnp.float32)


def _combine_body(x_ref, sguw_ref, sdwt_ref, b0_ref, b1_ref, w1_ref, w2_ref,
                  out_ref):
    xb = x_ref[...].astype(jnp.bfloat16)
    gu = jnp.dot(xb, sguw_ref[...], preferred_element_type=jnp.float32)
    act = _sigmoid(gu[:, :SHARED_DIM]) * gu[:, :SHARED_DIM] * gu[:, SHARED_DIM:]
    outs = jnp.dot(act.astype(jnp.bfloat16), sdwt_ref[...],
                   preferred_element_type=jnp.float32)
    out_ref[...] = (outs + w1_ref[...] * b0_ref[...].astype(jnp.float32)
                    + w2_ref[...] * b1_ref[...].astype(jnp.float32))


def _make_row_gather(n_rows_table, n_rows_out, n_cols, dtype):
    info = plsc.get_sparse_core_info()
    nw = info.num_cores * info.num_subcores
    b_per_w = n_rows_out // nw
    ch = 16
    while b_per_w % ch:
        ch //= 2
    nch = b_per_w // ch
    mesh = plsc.VectorSubcoreMesh(core_axis_name="c", subcore_axis_name="s")

    @functools.partial(
        pl.kernel,
        out_type=jax.ShapeDtypeStruct((n_rows_out, n_cols), dtype),
        mesh=mesh,
        scratch_types=[
            pltpu.VMEM((b_per_w,), jnp.int32),
            pltpu.VMEM((ch, n_cols), dtype),
            pltpu.VMEM((ch, n_cols), dtype),
            pltpu.SemaphoreType.DMA,
            pltpu.SemaphoreType.DMA,
        ],
    )
    def gather(table_hbm, idx_hbm, out_hbm, idx_v, rows0, rows1, sem0, sem1):
        wid = lax.axis_index("s") * info.num_cores + lax.axis_index("c")
        base = wid * b_per_w
        pltpu.sync_copy(idx_hbm.at[pl.ds(base, b_per_w)], idx_v)
        bufs = (rows0, rows1)
        sems = (sem0, sem1)
        handles = [None] * nch
        handles[0] = pltpu.async_copy(
            table_hbm.at[idx_v.at[pl.ds(0, ch)]], bufs[0], sems[0])
        for c in range(nch):
            if c + 1 < nch:
                handles[c + 1] = pltpu.async_copy(
                    table_hbm.at[idx_v.at[pl.ds((c + 1) * ch, ch)]],
                    bufs[(c + 1) % 2], sems[(c + 1) % 2])
            handles[c].wait()
            pltpu.sync_copy(bufs[c % 2], out_hbm.at[pl.ds(base + c * ch, ch)])

    return gather


def kernel(x, router_w, gate_up_w, down_w, shared_gate_w, shared_up_w,
           shared_down_w):
    B, T, C = x.shape
    N = B * T
    P = N * 2
    P_max = P + NUM_EXPERTS * BR
    G = P_max // BR
    x_flat = x.reshape(N, C)
    rwt = router_w.T
    guw16 = gate_up_w.astype(jnp.bfloat16)
    dw16 = down_w.astype(jnp.bfloat16)
    sguw = jnp.concatenate([shared_gate_w.T, shared_up_w.T], axis=1).astype(jnp.bfloat16)
    sdwt = shared_down_w.T.astype(jnp.bfloat16)

    BTR = 512
    w_dense = pl.pallas_call(
        _router_body,
        grid=(N // BTR,),
        in_specs=[
            pl.BlockSpec((BTR, C), lambda i: (i, 0)),
            pl.BlockSpec((C, NUM_EXPERTS), lambda i: (0, 0)),
        ],
        out_specs=pl.BlockSpec((BTR, NUM_EXPERTS), lambda i: (i, 0)),
        out_shape=jax.ShapeDtypeStruct((N, NUM_EXPERTS), jnp.float32),
    )(x_flat, rwt)

    eye = jnp.arange(NUM_EXPERTS, dtype=jnp.int32)
    w1 = jnp.max(w_dense, axis=1)
    e1 = jnp.argmax(w_dense, axis=1).astype(jnp.int32)
    wd2 = jnp.where(eye[None, :] == e1[:, None], -1.0, w_dense)
    w2 = jnp.max(wd2, axis=1)
    e2 = jnp.argmax(wd2, axis=1).astype(jnp.int32)
    sel = jnp.stack([e1, e2], axis=1).reshape(-1)
    order = jnp.argsort(sel, stable=True).astype(jnp.int32)
    sorted_e = sel[order]
    counts = jnp.sum((sel[:, None] == eye[None, :]).astype(jnp.int32), axis=0)
    offsets = jnp.concatenate([jnp.zeros(1, jnp.int32), jnp.cumsum(counts)[:-1]])
    pcounts = ((counts + BR - 1) // BR) * BR
    pcum = jnp.cumsum(pcounts)
    poffsets = jnp.concatenate([jnp.zeros(1, jnp.int32), pcum[:-1]])
    shift = (poffsets - offsets).astype(jnp.int32)
    pos_sorted = jnp.arange(P, dtype=jnp.int32) + shift[sorted_e]
    tok_padded = jnp.zeros(P_max, jnp.int32).at[pos_sorted].set(
        order // 2, mode="drop")
    block_expert = jnp.minimum(
        jnp.sum((jnp.arange(G, dtype=jnp.int32)[:, None] * BR >= pcum[None, :])
                .astype(jnp.int32), axis=1),
        NUM_EXPERTS - 1).astype(jnp.int32)
    inv = jnp.argsort(order).astype(jnp.int32)
    pos_unsorted = inv + shift[sel]
    p_cat = jnp.concatenate([pos_unsorted[0::2], pos_unsorted[1::2]])

    dispatch = _make_row_gather(N, P_max, C, jnp.float32)(x_flat, tok_padded)

    grid_spec = pltpu.PrefetchScalarGridSpec(
        num_scalar_prefetch=1,
        grid=(G,),
        in_specs=[
            pl.BlockSpec((BR, C), lambda g, be: (g, 0)),
            pl.BlockSpec((1, C, 2 * EXPERT_DIM), lambda g, be: (be[g], 0, 0)),
            pl.BlockSpec((1, EXPERT_DIM, C), lambda g, be: (be[g], 0, 0)),
        ],
        out_specs=pl.BlockSpec((BR, C), lambda g, be: (g, 0)),
    )
    out_sorted = pl.pallas_call(
        _expert_body,
        grid_spec=grid_spec,
        out_shape=jax.ShapeDtypeStruct((P_max, C), jnp.float32),
        compiler_params=pltpu.CompilerParams(
            dimension_semantics=("arbitrary",)),
    )(block_expert, dispatch, guw16, dw16)

    bufs = _make_row_gather(P_max, P, C, jnp.float32)(out_sorted, p_cat)

    BT2 = 256
    nb2 = N // BT2
    out = pl.pallas_call(
        _combine_body,
        grid=(nb2,),
        in_specs=[
            pl.BlockSpec((BT2, C), lambda i: (i, 0)),
            pl.BlockSpec((C, 2 * SHARED_DIM), lambda i: (0, 0)),
            pl.BlockSpec((SHARED_DIM, C), lambda i: (0, 0)),
            pl.BlockSpec((BT2, C), lambda i: (i, 0)),
            pl.BlockSpec((BT2, C), lambda i, _n=nb2: (i + _n, 0)),
            pl.BlockSpec((BT2, 1), lambda i: (i, 0)),
            pl.BlockSpec((BT2, 1), lambda i: (i, 0)),
        ],
        out_specs=pl.BlockSpec((BT2, C), lambda i: (i, 0)),
        out_shape=jax.ShapeDtypeStruct((N, C), jnp.float32),
    )(x_flat, sguw, sdwt, bufs, bufs, w1.reshape(N, 1), w2.reshape(N, 1))
    return out.reshape(B, T, C)


# speedup vs baseline: 2.6995x; 1.0193x over previous
import functools

import jax
import jax.numpy as jnp
from jax import lax
from jax.experimental import pallas as pl
from jax.experimental.pallas import tpu as pltpu
from jax.experimental.pallas import tpu_sc as plsc

D_MODEL = 2048
NUM_EXPERTS = 8
EXPERT_DIM = 1024
SHARED_DIM = 2048
BR = 256


def _sigmoid(z):
    return 1.0 / (1.0 + jnp.exp(-z))


def _router_body(x_ref, rwt_ref, w_ref):
    xf = x_ref[...]
    logits = jnp.dot(xf, rwt_ref[...], preferred_element_type=jnp.float32)
    iota = jax.lax.broadcasted_iota(jnp.int32, logits.shape, 1)
    big = jnp.int32(2**30)
    m1 = jnp.max(logits, axis=1, keepdims=True)
    a1 = jnp.min(jnp.where(logits == m1, iota, big), axis=1, keepdims=True)
    masked = jnp.where(iota == a1, -jnp.inf, logits)
    m2 = jnp.max(masked, axis=1, keepdims=True)
    a2 = jnp.min(jnp.where(masked == m2, iota, big), axis=1, keepdims=True)
    w_ref[...] = jnp.where(iota == a1, _sigmoid(m1), 0.0) + jnp.where(
        iota == a2, _sigmoid(m2), 0.0)


def _expert_body(meta_ref, disp_ref, guw_ref, dw_ref, out_ref):
    g = pl.program_id(0)

    @pl.when(g * BR < meta_ref[pl.num_programs(0)])
    def _():
        xb = disp_ref[...].astype(jnp.bfloat16)
        gu = jnp.dot(xb, guw_ref[0], preferred_element_type=jnp.float32)
        act = (_sigmoid(gu[:, :EXPERT_DIM]) * gu[:, :EXPERT_DIM]
               * gu[:, EXPERT_DIM:])
        out_ref[...] = jnp.dot(act.astype(jnp.bfloat16), dw_ref[0],
                               preferred_element_type=jnp.float32)


def _combine_body(x_ref, sguw_ref, sdwt_ref, b0_ref, b1_ref, w1_ref, w2_ref,
                  out_ref):
    xb = x_ref[...].astype(jnp.bfloat16)
    gu = jnp.dot(xb, sguw_ref[...], preferred_element_type=jnp.float32)
    act = _sigmoid(gu[:, :SHARED_DIM]) * gu[:, :SHARED_DIM] * gu[:, SHARED_DIM:]
    outs = jnp.dot(act.astype(jnp.bfloat16), sdwt_ref[...],
                   preferred_element_type=jnp.float32)
    out_ref[...] = (outs + w1_ref[...] * b0_ref[...].astype(jnp.float32)
                    + w2_ref[...] * b1_ref[...].astype(jnp.float32))


def _make_row_gather(n_rows_table, n_rows_out, n_cols, dtype):
    info = plsc.get_sparse_core_info()
    nw = info.num_cores * info.num_subcores
    b_per_w = n_rows_out // nw
    ch = 16
    while b_per_w % ch:
        ch //= 2
    nch = b_per_w // ch
    mesh = plsc.VectorSubcoreMesh(core_axis_name="c", subcore_axis_name="s")

    @functools.partial(
        pl.kernel,
        out_type=jax.ShapeDtypeStruct((n_rows_out, n_cols), dtype),
        mesh=mesh,
        scratch_types=[
            pltpu.VMEM((b_per_w,), jnp.int32),
            pltpu.VMEM((ch, n_cols), dtype),
            pltpu.VMEM((ch, n_cols), dtype),
            pltpu.SemaphoreType.DMA,
            pltpu.SemaphoreType.DMA,
        ],
    )
    def gather(table_hbm, idx_hbm, out_hbm, idx_v, rows0, rows1, sem0, sem1):
        wid = lax.axis_index("s") * info.num_cores + lax.axis_index("c")
        base = wid * b_per_w
        pltpu.sync_copy(idx_hbm.at[pl.ds(base, b_per_w)], idx_v)
        bufs = (rows0, rows1)
        sems = (sem0, sem1)
        handles = [None] * nch
        handles[0] = pltpu.async_copy(
            table_hbm.at[idx_v.at[pl.ds(0, ch)]], bufs[0], sems[0])
        for c in range(nch):
            if c + 1 < nch:
                handles[c + 1] = pltpu.async_copy(
                    table_hbm.at[idx_v.at[pl.ds((c + 1) * ch, ch)]],
                    bufs[(c + 1) % 2], sems[(c + 1) % 2])
            handles[c].wait()
            pltpu.sync_copy(bufs[c % 2], out_hbm.at[pl.ds(base + c * ch, ch)])

    return gather


def kernel(x, router_w, gate_up_w, down_w, shared_gate_w, shared_up_w,
           shared_down_w):
    B, T, C = x.shape
    N = B * T
    P = N * 2
    P_max = P + NUM_EXPERTS * BR
    G = P_max // BR
    x_flat = x.reshape(N, C)
    rwt = router_w.T
    guw16 = gate_up_w.astype(jnp.bfloat16)
    dw16 = down_w.astype(jnp.bfloat16)
    sguw = jnp.concatenate([shared_gate_w.T, shared_up_w.T], axis=1).astype(jnp.bfloat16)
    sdwt = shared_down_w.T.astype(jnp.bfloat16)

    BTR = 512
    w_dense = pl.pallas_call(
        _router_body,
        grid=(N // BTR,),
        in_specs=[
            pl.BlockSpec((BTR, C), lambda i: (i, 0)),
            pl.BlockSpec((C, NUM_EXPERTS), lambda i: (0, 0)),
        ],
        out_specs=pl.BlockSpec((BTR, NUM_EXPERTS), lambda i: (i, 0)),
        out_shape=jax.ShapeDtypeStruct((N, NUM_EXPERTS), jnp.float32),
    )(x_flat, rwt)

    eye = jnp.arange(NUM_EXPERTS, dtype=jnp.int32)
    w1 = jnp.max(w_dense, axis=1)
    e1 = jnp.argmax(w_dense, axis=1).astype(jnp.int32)
    wd2 = jnp.where(eye[None, :] == e1[:, None], -1.0, w_dense)
    w2 = jnp.max(wd2, axis=1)
    e2 = jnp.argmax(wd2, axis=1).astype(jnp.int32)
    sel = jnp.stack([e1, e2], axis=1).reshape(-1)
    oh = sel[:, None] == eye[None, :]
    rank_incl = jnp.cumsum(oh.astype(jnp.int32), axis=0)
    rank = jnp.sum(jnp.where(oh, rank_incl, 0), axis=1) - 1
    counts = rank_incl[-1]
    pcounts = ((counts + BR - 1) // BR) * BR
    pcum = jnp.cumsum(pcounts)
    poffsets = (pcum - pcounts).astype(jnp.int32)
    pos = poffsets[sel] + rank
    tok_padded = jnp.zeros(P_max, jnp.int32).at[pos].set(
        jnp.arange(P, dtype=jnp.int32) // 2, mode="drop", unique_indices=True)
    block_expert = jnp.minimum(
        jnp.sum((jnp.arange(G, dtype=jnp.int32)[:, None] * BR >= pcum[None, :])
                .astype(jnp.int32), axis=1),
        NUM_EXPERTS - 1).astype(jnp.int32)
    meta = jnp.concatenate([block_expert, pcum[-1:].astype(jnp.int32)])
    p_cat = jnp.concatenate([pos[0::2], pos[1::2]])

    dispatch = _make_row_gather(N, P_max, C, jnp.float32)(x_flat, tok_padded)

    grid_spec = pltpu.PrefetchScalarGridSpec(
        num_scalar_prefetch=1,
        grid=(G,),
        in_specs=[
            pl.BlockSpec((BR, C), lambda g, be: (g, 0)),
            pl.BlockSpec((1, C, 2 * EXPERT_DIM), lambda g, be: (be[g], 0, 0)),
            pl.BlockSpec((1, EXPERT_DIM, C), lambda g, be: (be[g], 0, 0)),
        ],
        out_specs=pl.BlockSpec((BR, C), lambda g, be: (g, 0)),
    )
    out_sorted = pl.pallas_call(
        _expert_body,
        grid_spec=grid_spec,
        out_shape=jax.ShapeDtypeStruct((P_max, C), jnp.float32),
        compiler_params=pltpu.CompilerParams(
            dimension_semantics=("arbitrary",)),
    )(meta, dispatch, guw16, dw16)

    bufs = _make_row_gather(P_max, P, C, jnp.float32)(out_sorted, p_cat)

    BT2 = 256
    nb2 = N // BT2
    out = pl.pallas_call(
        _combine_body,
        grid=(nb2,),
        in_specs=[
            pl.BlockSpec((BT2, C), lambda i: (i, 0)),
            pl.BlockSpec((C, 2 * SHARED_DIM), lambda i: (0, 0)),
            pl.BlockSpec((SHARED_DIM, C), lambda i: (0, 0)),
            pl.BlockSpec((BT2, C), lambda i: (i, 0)),
            pl.BlockSpec((BT2, C), lambda i, _n=nb2: (i + _n, 0)),
            pl.BlockSpec((BT2, 1), lambda i: (i, 0)),
            pl.BlockSpec((BT2, 1), lambda i: (i, 0)),
        ],
        out_specs=pl.BlockSpec((BT2, C), lambda i: (i, 0)),
        out_shape=jax.ShapeDtypeStruct((N, C), jnp.float32),
    )(x_flat, sguw, sdwt, bufs, bufs, w1.reshape(N, 1), w2.reshape(N, 1))
    return out.reshape(B, T, C)


# speedup vs baseline: 2.7878x; 1.0327x over previous
import functools

import jax
import jax.numpy as jnp
from jax import lax
from jax.experimental import pallas as pl
from jax.experimental.pallas import tpu as pltpu
from jax.experimental.pallas import tpu_sc as plsc

D_MODEL = 2048
NUM_EXPERTS = 8
EXPERT_DIM = 1024
SHARED_DIM = 2048
BR = 256


def _sigmoid(z):
    return 1.0 / (1.0 + jnp.exp(-z))


def _rnd_bf16_bits(v):
    u = lax.bitcast_convert_type(v, jnp.uint32)
    return (u + jnp.uint32(0x7FFF) + ((u >> 16) & jnp.uint32(1))) >> 16


def _pack_halves(a_f32, b_f32):
    return lax.bitcast_convert_type(
        _rnd_bf16_bits(a_f32) | (_rnd_bf16_bits(b_f32) << 16), jnp.int32)


def _unpack_halves(p_i32):
    u = lax.bitcast_convert_type(p_i32, jnp.uint32)
    lo = lax.bitcast_convert_type(u << 16, jnp.float32)
    hi = lax.bitcast_convert_type(u & jnp.uint32(0xFFFF0000), jnp.float32)
    return lo, hi


def _router_body(x_ref, rwt_ref, w_ref, xp_ref):
    xf = x_ref[...]
    half = xf.shape[1] // 2
    xp_ref[...] = _pack_halves(xf[:, :half], xf[:, half:])
    logits = jnp.dot(xf, rwt_ref[...], preferred_element_type=jnp.float32)
    iota = jax.lax.broadcasted_iota(jnp.int32, logits.shape, 1)
    big = jnp.int32(2**30)
    m1 = jnp.max(logits, axis=1, keepdims=True)
    a1 = jnp.min(jnp.where(logits == m1, iota, big), axis=1, keepdims=True)
    masked = jnp.where(iota == a1, -jnp.inf, logits)
    m2 = jnp.max(masked, axis=1, keepdims=True)
    a2 = jnp.min(jnp.where(masked == m2, iota, big), axis=1, keepdims=True)
    w_ref[...] = jnp.where(iota == a1, _sigmoid(m1), 0.0) + jnp.where(
        iota == a2, _sigmoid(m2), 0.0)


def _expert_body(meta_ref, disp_ref, guw_ref, dw_ref, out_ref):
    g = pl.program_id(0)

    @pl.when(g * BR < meta_ref[pl.num_programs(0)])
    def _():
        half = D_MODEL // 2
        xlo, xhi = _unpack_halves(disp_ref[...])
        gu = jnp.dot(xlo.astype(jnp.bfloat16), guw_ref[0, :half],
                     preferred_element_type=jnp.float32)
        gu += jnp.dot(xhi.astype(jnp.bfloat16), guw_ref[0, half:],
                      preferred_element_type=jnp.float32)
        act = (_sigmoid(gu[:, :EXPERT_DIM]) * gu[:, :EXPERT_DIM]
               * gu[:, EXPERT_DIM:])
        o = jnp.dot(act.astype(jnp.bfloat16), dw_ref[0],
                    preferred_element_type=jnp.float32)
        out_ref[...] = _pack_halves(o[:, :half], o[:, half:])


def _shared_body(x_ref, sguw_ref, sdwt_ref, out_ref):
    xb = x_ref[...].astype(jnp.bfloat16)
    gu = jnp.dot(xb, sguw_ref[...], preferred_element_type=jnp.float32)
    act = _sigmoid(gu[:, :SHARED_DIM]) * gu[:, :SHARED_DIM] * gu[:, SHARED_DIM:]
    out_ref[...] = jnp.dot(act.astype(jnp.bfloat16), sdwt_ref[...],
                           preferred_element_type=jnp.float32)


def _combine_body(sh_ref, b0_ref, b1_ref, w1_ref, w2_ref, out_ref):
    half = D_MODEL // 2
    a0, b0 = _unpack_halves(b0_ref[...])
    a1, b1 = _unpack_halves(b1_ref[...])
    w1 = w1_ref[...]
    w2 = w2_ref[...]
    sh = sh_ref[...]
    out_ref[:, :half] = sh[:, :half] + w1 * a0 + w2 * a1
    out_ref[:, half:] = sh[:, half:] + w1 * b0 + w2 * b1


def _make_row_gather(n_rows_table, n_rows_out, n_cols, dtype):
    info = plsc.get_sparse_core_info()
    nw = info.num_cores * info.num_subcores
    b_per_w = n_rows_out // nw
    ch = 32 if n_cols <= 1024 else 16
    while b_per_w % ch:
        ch //= 2
    nch = b_per_w // ch
    mesh = plsc.VectorSubcoreMesh(core_axis_name="c", subcore_axis_name="s")

    @functools.partial(
        pl.kernel,
        out_type=jax.ShapeDtypeStruct((n_rows_out, n_cols), dtype),
        mesh=mesh,
        scratch_types=[
            pltpu.VMEM((b_per_w,), jnp.int32),
            pltpu.VMEM((ch, n_cols), dtype),
            pltpu.VMEM((ch, n_cols), dtype),
            pltpu.SemaphoreType.DMA,
            pltpu.SemaphoreType.DMA,
        ],
    )
    def gather(table_hbm, idx_hbm, out_hbm, idx_v, rows0, rows1, sem0, sem1):
        wid = lax.axis_index("s") * info.num_cores + lax.axis_index("c")
        base = wid * b_per_w
        pltpu.sync_copy(idx_hbm.at[pl.ds(base, b_per_w)], idx_v)
        bufs = (rows0, rows1)
        sems = (sem0, sem1)
        handles = [None] * nch
        handles[0] = pltpu.async_copy(
            table_hbm.at[idx_v.at[pl.ds(0, ch)]], bufs[0], sems[0])
        for c in range(nch):
            if c + 1 < nch:
                handles[c + 1] = pltpu.async_copy(
                    table_hbm.at[idx_v.at[pl.ds((c + 1) * ch, ch)]],
                    bufs[(c + 1) % 2], sems[(c + 1) % 2])
            handles[c].wait()
            pltpu.sync_copy(bufs[c % 2], out_hbm.at[pl.ds(base + c * ch, ch)])

    return gather


def kernel(x, router_w, gate_up_w, down_w, shared_gate_w, shared_up_w,
           shared_down_w):
    B, T, C = x.shape
    N = B * T
    P = N * 2
    P_max = P + NUM_EXPERTS * BR
    G = P_max // BR
    x_flat = x.reshape(N, C)
    rwt = router_w.T
    guw16 = gate_up_w.astype(jnp.bfloat16)
    dw16 = down_w.astype(jnp.bfloat16)
    sguw = jnp.concatenate([shared_gate_w.T, shared_up_w.T], axis=1).astype(jnp.bfloat16)
    sdwt = shared_down_w.T.astype(jnp.bfloat16)

    BTR = 512
    H = C // 2
    w_dense, x_packed = pl.pallas_call(
        _router_body,
        grid=(N // BTR,),
        in_specs=[
            pl.BlockSpec((BTR, C), lambda i: (i, 0)),
            pl.BlockSpec((C, NUM_EXPERTS), lambda i: (0, 0)),
        ],
        out_specs=[
            pl.BlockSpec((BTR, NUM_EXPERTS), lambda i: (i, 0)),
            pl.BlockSpec((BTR, H), lambda i: (i, 0)),
        ],
        out_shape=[
            jax.ShapeDtypeStruct((N, NUM_EXPERTS), jnp.float32),
            jax.ShapeDtypeStruct((N, H), jnp.int32),
        ],
    )(x_flat, rwt)

    eye = jnp.arange(NUM_EXPERTS, dtype=jnp.int32)
    w1 = jnp.max(w_dense, axis=1)
    e1 = jnp.argmax(w_dense, axis=1).astype(jnp.int32)
    wd2 = jnp.where(eye[None, :] == e1[:, None], -1.0, w_dense)
    w2 = jnp.max(wd2, axis=1)
    e2 = jnp.argmax(wd2, axis=1).astype(jnp.int32)
    sel = jnp.stack([e1, e2], axis=1).reshape(-1)
    oh = sel[:, None] == eye[None, :]
    rank_incl = jnp.cumsum(oh.astype(jnp.int32), axis=0)
    rank = jnp.sum(jnp.where(oh, rank_incl, 0), axis=1) - 1
    counts = rank_incl[-1]
    pcounts = ((counts + BR - 1) // BR) * BR
    pcum = jnp.cumsum(pcounts)
    poffsets = (pcum - pcounts).astype(jnp.int32)
    pos = poffsets[sel] + rank
    tok_padded = jnp.zeros(P_max, jnp.int32).at[pos].set(
        jnp.arange(P, dtype=jnp.int32) // 2, mode="drop", unique_indices=True)
    block_expert = jnp.minimum(
        jnp.sum((jnp.arange(G, dtype=jnp.int32)[:, None] * BR >= pcum[None, :])
                .astype(jnp.int32), axis=1),
        NUM_EXPERTS - 1).astype(jnp.int32)
    meta = jnp.concatenate([block_expert, pcum[-1:].astype(jnp.int32)])
    p_cat = jnp.concatenate([pos[0::2], pos[1::2]])

    dispatch = _make_row_gather(N, P_max, H, jnp.int32)(x_packed, tok_padded)

    BT2 = 512
    shared_out = pl.pallas_call(
        _shared_body,
        grid=(N // BT2,),
        in_specs=[
            pl.BlockSpec((BT2, C), lambda i: (i, 0)),
            pl.BlockSpec((C, 2 * SHARED_DIM), lambda i: (0, 0)),
            pl.BlockSpec((SHARED_DIM, C), lambda i: (0, 0)),
        ],
        out_specs=pl.BlockSpec((BT2, C), lambda i: (i, 0)),
        out_shape=jax.ShapeDtypeStruct((N, C), jnp.float32),
    )(x_flat, sguw, sdwt)

    grid_spec = pltpu.PrefetchScalarGridSpec(
        num_scalar_prefetch=1,
        grid=(G,),
        in_specs=[
            pl.BlockSpec((BR, H), lambda g, be: (g, 0)),
            pl.BlockSpec((1, C, 2 * EXPERT_DIM), lambda g, be: (be[g], 0, 0)),
            pl.BlockSpec((1, EXPERT_DIM, C), lambda g, be: (be[g], 0, 0)),
        ],
        out_specs=pl.BlockSpec((BR, H), lambda g, be: (g, 0)),
    )
    out_sorted = pl.pallas_call(
        _expert_body,
        grid_spec=grid_spec,
        out_shape=jax.ShapeDtypeStruct((P_max, H), jnp.int32),
        compiler_params=pltpu.CompilerParams(
            dimension_semantics=("arbitrary",)),
    )(meta, dispatch, guw16, dw16)

    bufs = _make_row_gather(P_max, P, H, jnp.int32)(out_sorted, p_cat)

    BT3 = 512
    nb3 = N // BT3
    out = pl.pallas_call(
        _combine_body,
        grid=(nb3,),
        in_specs=[
            pl.BlockSpec((BT3, C), lambda i: (i, 0)),
            pl.BlockSpec((BT3, H), lambda i: (i, 0)),
            pl.BlockSpec((BT3, H), lambda i, _n=nb3: (i + _n, 0)),
            pl.BlockSpec((BT3, 1), lambda i: (i, 0)),
            pl.BlockSpec((BT3, 1), lambda i: (i, 0)),
        ],
        out_specs=pl.BlockSpec((BT3, C), lambda i: (i, 0)),
        out_shape=jax.ShapeDtypeStruct((N, C), jnp.float32),
    )(shared_out, bufs, bufs, w1.reshape(N, 1), w2.reshape(N, 1))
    return out.reshape(B, T, C)


# speedup vs baseline: 2.9085x; 1.0433x over previous
import functools

import jax
import jax.numpy as jnp
from jax import lax
from jax.experimental import pallas as pl
from jax.experimental.pallas import tpu as pltpu
from jax.experimental.pallas import tpu_sc as plsc

D_MODEL = 2048
NUM_EXPERTS = 8
EXPERT_DIM = 1024
SHARED_DIM = 2048
BR = 256


def _sigmoid(z):
    return 1.0 / (1.0 + jnp.exp(-z))


def _rnd_bf16_bits(v):
    u = lax.bitcast_convert_type(v, jnp.uint32)
    return (u + jnp.uint32(0x7FFF) + ((u >> 16) & jnp.uint32(1))) >> 16


def _pack_halves(a_f32, b_f32):
    return lax.bitcast_convert_type(
        _rnd_bf16_bits(a_f32) | (_rnd_bf16_bits(b_f32) << 16), jnp.int32)


def _unpack_halves(p_i32):
    u = lax.bitcast_convert_type(p_i32, jnp.uint32)
    lo = lax.bitcast_convert_type(u << 16, jnp.float32)
    hi = lax.bitcast_convert_type(u & jnp.uint32(0xFFFF0000), jnp.float32)
    return lo, hi


def _router_body(x_ref, rwt_ref, w_ref, xp_ref):
    xf = x_ref[...]
    half = xf.shape[1] // 2
    xp_ref[...] = _pack_halves(xf[:, :half], xf[:, half:])
    logits = jnp.dot(xf, rwt_ref[...], preferred_element_type=jnp.float32)
    iota = jax.lax.broadcasted_iota(jnp.int32, logits.shape, 1)
    big = jnp.int32(2**30)
    m1 = jnp.max(logits, axis=1, keepdims=True)
    a1 = jnp.min(jnp.where(logits == m1, iota, big), axis=1, keepdims=True)
    masked = jnp.where(iota == a1, -jnp.inf, logits)
    m2 = jnp.max(masked, axis=1, keepdims=True)
    a2 = jnp.min(jnp.where(masked == m2, iota, big), axis=1, keepdims=True)
    w_ref[...] = jnp.where(iota == a1, _sigmoid(m1), 0.0) + jnp.where(
        iota == a2, _sigmoid(m2), 0.0)


def _expert_body(meta_ref, disp_ref, guw_ref, dw_ref, out_ref):
    g = pl.program_id(0)

    @pl.when(g * BR < meta_ref[pl.num_programs(0)])
    def _():
        half = D_MODEL // 2
        xlo, xhi = _unpack_halves(disp_ref[...])
        gu = jnp.dot(xlo.astype(jnp.bfloat16), guw_ref[0, :half],
                     preferred_element_type=jnp.float32)
        gu += jnp.dot(xhi.astype(jnp.bfloat16), guw_ref[0, half:],
                      preferred_element_type=jnp.float32)
        act = (_sigmoid(gu[:, :EXPERT_DIM]) * gu[:, :EXPERT_DIM]
               * gu[:, EXPERT_DIM:])
        o = jnp.dot(act.astype(jnp.bfloat16), dw_ref[0],
                    preferred_element_type=jnp.float32)
        out_ref[...] = _pack_halves(o[:, :half], o[:, half:])


def _combine_body(x_ref, sguw_ref, sdwt_ref, b0_ref, b1_ref, w1_ref, w2_ref,
                  out_ref):
    xb = x_ref[...].astype(jnp.bfloat16)
    gu = jnp.dot(xb, sguw_ref[...], preferred_element_type=jnp.float32)
    act = _sigmoid(gu[:, :SHARED_DIM]) * gu[:, :SHARED_DIM] * gu[:, SHARED_DIM:]
    sh = jnp.dot(act.astype(jnp.bfloat16), sdwt_ref[...],
                 preferred_element_type=jnp.float32)
    half = D_MODEL // 2
    a0, b0 = _unpack_halves(b0_ref[...])
    a1, b1 = _unpack_halves(b1_ref[...])
    w1 = w1_ref[...]
    w2 = w2_ref[...]
    out_ref[:, :half] = sh[:, :half] + w1 * a0 + w2 * a1
    out_ref[:, half:] = sh[:, half:] + w1 * b0 + w2 * b1


def _make_row_gather(n_rows_table, n_rows_out, n_cols, dtype):
    info = plsc.get_sparse_core_info()
    nw = info.num_cores * info.num_subcores
    b_per_w = n_rows_out // nw
    ch = 32 if n_cols <= 1024 else 16
    while b_per_w % ch:
        ch //= 2
    nch = b_per_w // ch
    mesh = plsc.VectorSubcoreMesh(core_axis_name="c", subcore_axis_name="s")

    @functools.partial(
        pl.kernel,
        out_type=jax.ShapeDtypeStruct((n_rows_out, n_cols), dtype),
        mesh=mesh,
        scratch_types=[
            pltpu.VMEM((b_per_w,), jnp.int32),
            pltpu.VMEM((ch, n_cols), dtype),
            pltpu.VMEM((ch, n_cols), dtype),
            pltpu.SemaphoreType.DMA,
            pltpu.SemaphoreType.DMA,
        ],
    )
    def gather(table_hbm, idx_hbm, out_hbm, idx_v, rows0, rows1, sem0, sem1):
        wid = lax.axis_index("s") * info.num_cores + lax.axis_index("c")
        base = wid * b_per_w
        pltpu.sync_copy(idx_hbm.at[pl.ds(base, b_per_w)], idx_v)
        bufs = (rows0, rows1)
        sems = (sem0, sem1)
        handles = [None] * nch
        handles[0] = pltpu.async_copy(
            table_hbm.at[idx_v.at[pl.ds(0, ch)]], bufs[0], sems[0])
        for c in range(nch):
            if c + 1 < nch:
                handles[c + 1] = pltpu.async_copy(
                    table_hbm.at[idx_v.at[pl.ds((c + 1) * ch, ch)]],
                    bufs[(c + 1) % 2], sems[(c + 1) % 2])
            handles[c].wait()
            pltpu.sync_copy(bufs[c % 2], out_hbm.at[pl.ds(base + c * ch, ch)])

    return gather


def kernel(x, router_w, gate_up_w, down_w, shared_gate_w, shared_up_w,
           shared_down_w):
    B, T, C = x.shape
    N = B * T
    P = N * 2
    P_max = P + NUM_EXPERTS * BR
    G = P_max // BR
    x_flat = x.reshape(N, C)
    rwt = router_w.T
    guw16 = gate_up_w.astype(jnp.bfloat16)
    dw16 = down_w.astype(jnp.bfloat16)
    sguw = jnp.concatenate([shared_gate_w.T, shared_up_w.T], axis=1).astype(jnp.bfloat16)
    sdwt = shared_down_w.T.astype(jnp.bfloat16)

    BTR = 512
    H = C // 2
    w_dense, x_packed = pl.pallas_call(
        _router_body,
        grid=(N // BTR,),
        in_specs=[
            pl.BlockSpec((BTR, C), lambda i: (i, 0)),
            pl.BlockSpec((C, NUM_EXPERTS), lambda i: (0, 0)),
        ],
        out_specs=[
            pl.BlockSpec((BTR, NUM_EXPERTS), lambda i: (i, 0)),
            pl.BlockSpec((BTR, H), lambda i: (i, 0)),
        ],
        out_shape=[
            jax.ShapeDtypeStruct((N, NUM_EXPERTS), jnp.float32),
            jax.ShapeDtypeStruct((N, H), jnp.int32),
        ],
    )(x_flat, rwt)

    eye = jnp.arange(NUM_EXPERTS, dtype=jnp.int32)
    w1 = jnp.max(w_dense, axis=1)
    e1 = jnp.argmax(w_dense, axis=1).astype(jnp.int32)
    wd2 = jnp.where(eye[None, :] == e1[:, None], -1.0, w_dense)
    w2 = jnp.max(wd2, axis=1)
    e2 = jnp.argmax(wd2, axis=1).astype(jnp.int32)
    sel = jnp.stack([e1, e2], axis=1).reshape(-1)
    oh = sel[:, None] == eye[None, :]
    rank_incl = jnp.cumsum(oh.astype(jnp.int32), axis=0)
    rank = jnp.sum(jnp.where(oh, rank_incl, 0), axis=1) - 1
    counts = rank_incl[-1]
    pcounts = ((counts + BR - 1) // BR) * BR
    pcum = jnp.cumsum(pcounts)
    poffsets = (pcum - pcounts).astype(jnp.int32)
    pos = poffsets[sel] + rank
    tok_padded = jnp.zeros(P_max, jnp.int32).at[pos].set(
        jnp.arange(P, dtype=jnp.int32) // 2, mode="drop", unique_indices=True)
    block_expert = jnp.minimum(
        jnp.sum((jnp.arange(G, dtype=jnp.int32)[:, None] * BR >= pcum[None, :])
                .astype(jnp.int32), axis=1),
        NUM_EXPERTS - 1).astype(jnp.int32)
    meta = jnp.concatenate([block_expert, pcum[-1:].astype(jnp.int32)])
    p_cat = jnp.concatenate([pos[0::2], pos[1::2]])

    dispatch = _make_row_gather(N, P_max, H, jnp.int32)(x_packed, tok_padded)

    grid_spec = pltpu.PrefetchScalarGridSpec(
        num_scalar_prefetch=1,
        grid=(G,),
        in_specs=[
            pl.BlockSpec((BR, H), lambda g, be: (g, 0)),
            pl.BlockSpec((1, C, 2 * EXPERT_DIM), lambda g, be: (be[g], 0, 0)),
            pl.BlockSpec((1, EXPERT_DIM, C), lambda g, be: (be[g], 0, 0)),
        ],
        out_specs=pl.BlockSpec((BR, H), lambda g, be: (g, 0)),
    )
    out_sorted = pl.pallas_call(
        _expert_body,
        grid_spec=grid_spec,
        out_shape=jax.ShapeDtypeStruct((P_max, H), jnp.int32),
        compiler_params=pltpu.CompilerParams(
            dimension_semantics=("arbitrary",)),
    )(meta, dispatch, guw16, dw16)

    bufs = _make_row_gather(P_max, P, H, jnp.int32)(out_sorted, p_cat)

    BT3 = 256
    nb3 = N // BT3
    out = pl.pallas_call(
        _combine_body,
        grid=(nb3,),
        in_specs=[
            pl.BlockSpec((BT3, C), lambda i: (i, 0)),
            pl.BlockSpec((C, 2 * SHARED_DIM), lambda i: (0, 0)),
            pl.BlockSpec((SHARED_DIM, C), lambda i: (0, 0)),
            pl.BlockSpec((BT3, H), lambda i: (i, 0)),
            pl.BlockSpec((BT3, H), lambda i, _n=nb3: (i + _n, 0)),
            pl.BlockSpec((BT3, 1), lambda i: (i, 0)),
            pl.BlockSpec((BT3, 1), lambda i: (i, 0)),
        ],
        out_specs=pl.BlockSpec((BT3, C), lambda i: (i, 0)),
        out_shape=jax.ShapeDtypeStruct((N, C), jnp.float32),
    )(x_flat, sguw, sdwt, bufs, bufs, w1.reshape(N, 1), w2.reshape(N, 1))
    return out.reshape(B, T, C)
